# baseline (speedup 1.0000x reference)
import dataclasses
import functools

import jax
import jax.numpy as jnp
from jax import lax
from jax.experimental import pallas as pl
from jax.experimental.pallas import tpu as pltpu
from jax.experimental.pallas import tpu_sc as plsc

N_NODES = 10000
N_EDGES = 320000
D = 128

NC = 2
NS = 16
NW = NC * NS
CHUNK = 128
GRP = 8
CPW = 80
EPW = CHUNK * CPW
E_PAD = EPW * NW
NBLK = CPW // GRP
N_PAD = 10112
RPW = N_PAD // NS
NZ = RPW // CHUNK
TAIL = RPW % CHUNK
DUMMY_ROW = N_NODES + 8

_MESH = plsc.VectorSubcoreMesh(core_axis_name="c", subcore_axis_name="s")


def _zero_stripe(zv, shared, base):
    for k in range(NZ):
        pltpu.sync_copy(zv, shared.at[pl.ds(base + k * CHUNK, CHUNK)])
    if TAIL:
        pltpu.sync_copy(zv.at[pl.ds(0, TAIL)],
                        shared.at[pl.ds(base + NZ * CHUNK, TAIL)])


def _sc_aggregate(x, ii, with_deg):
    def body(x_hbm, ii_hbm, z128_hbm, *rest):
        if with_deg:
            (agg_out, deg_out, agg_sh, idx_v, r0, r1,
             hist, sg0, sg1, ss0, ss1) = rest
        else:
            (agg_out, agg_sh, idx_v, r0, r1, sg0, sg1, ss0, ss1) = rest
        c = lax.axis_index("c")
        s = lax.axis_index("s")
        w = c * NS + s
        base = s * RPW

        pltpu.sync_copy(z128_hbm, r0)
        _zero_stripe(r0, agg_sh, base)
        if with_deg:
            zeros16 = jnp.zeros((16,), jnp.float32)

            @pl.loop(0, N_PAD // 16)
            def _(i):
                hist[pl.ds(i * 16, 16)] = zeros16
        plsc.subcore_barrier()

        ones16 = jnp.ones((16,), jnp.float32)

        @pl.loop(0, NBLK)
        def _(t):
            pltpu.sync_copy(ii_hbm.at[w * NBLK + t], idx_v)
            for p0 in range(0, GRP, 2):
                p1 = p0 + 1
                g0 = pltpu.async_copy(x_hbm.at[idx_v.at[0, p0]], r0, sg0)
                g1 = pltpu.async_copy(x_hbm.at[idx_v.at[0, p1]], r1, sg1)
                if with_deg:
                    for b in (p0, p1):
                        for j in range(CHUNK // 16):
                            idxv = idx_v[1, b, pl.ds(j * 16, 16)]
                            plsc.addupdate_scatter(hist, [idxv], ones16)
                g0.wait()
                s0 = pltpu.async_copy(r0, agg_sh.at[idx_v.at[1, p0]], ss0,
                                      add=True)
                g1.wait()
                s1 = pltpu.async_copy(r1, agg_sh.at[idx_v.at[1, p1]], ss1,
                                      add=True)
                s0.wait()
                s1.wait()

        plsc.subcore_barrier()
        pltpu.sync_copy(agg_sh.at[pl.ds(base, RPW)],
                        agg_out.at[pl.ds(c * N_PAD + base, RPW)])
        if with_deg:
            pltpu.sync_copy(hist, deg_out.at[w])

    out_type = [jax.ShapeDtypeStruct((NC * N_PAD, D), jnp.float32)]
    scratch = [
        pltpu.VMEM_SHARED((N_PAD, D), jnp.float32),
        pltpu.VMEM((2, GRP, CHUNK), jnp.int32),
        pltpu.VMEM((CHUNK, D), jnp.float32),
        pltpu.VMEM((CHUNK, D), jnp.float32),
    ]
    if with_deg:
        out_type.append(jax.ShapeDtypeStruct((NW, N_PAD), jnp.float32))
        scratch.append(pltpu.VMEM((N_PAD,), jnp.float32))
    scratch += [pltpu.SemaphoreType.DMA] * 4

    cp = pltpu.CompilerParams()
    if "needs_layout_passes" in pltpu.CompilerParams.__dataclass_fields__:
        cp = dataclasses.replace(cp, needs_layout_passes=False)
    k = pl.kernel(body, out_type=tuple(out_type), mesh=_MESH,
                  compiler_params=cp, scratch_types=scratch)
    return k(x, ii, jnp.zeros((CHUNK, D), jnp.float32))


def _tc_layer(aggp, degp, x, Wl, bl, Wr, relu):
    def body(aggp_ref, degp_ref, ones_ref, x_ref, wl_ref, bl_ref, wr_ref,
             o_ref):
        agg = aggp_ref[:N_NODES, :] + aggp_ref[N_PAD:N_PAD + N_NODES, :]
        deg = lax.dot_general(degp_ref[:, :N_NODES], ones_ref[...],
                              (((0,), (0,)), ((), ())),
                              preferred_element_type=jnp.float32)
        mean = agg / jnp.maximum(deg, 1.0)
        h = lax.dot_general(mean, wl_ref[...], (((1,), (1,)), ((), ())),
                            preferred_element_type=jnp.float32)
        h = h + bl_ref[...]
        h = h + lax.dot_general(x_ref[...], wr_ref[...],
                                (((1,), (1,)), ((), ())),
                                preferred_element_type=jnp.float32)
        o_ref[...] = jnp.maximum(h, 0.0) if relu else h

    return pl.pallas_call(
        body,
        out_shape=jax.ShapeDtypeStruct((N_NODES, D), jnp.float32),
    )(aggp, degp, jnp.ones((NW, 1), jnp.float32), x, Wl, bl, Wr)


@jax.jit
def kernel(x, edge_index, Wl1, bl1, Wr1, Wl2, bl2, Wr2):
    src = edge_index[0].astype(jnp.int32)
    dst = edge_index[1].astype(jnp.int32)
    pad = E_PAD - N_EDGES
    src = jnp.concatenate([src, jnp.zeros((pad,), jnp.int32)])
    dst = jnp.concatenate([dst, jnp.full((pad,), DUMMY_ROW, jnp.int32)])
    si = src.reshape(NW * NBLK, GRP, CHUNK)
    di = dst.reshape(NW * NBLK, GRP, CHUNK)
    ii = jnp.stack([si, di], axis=1)

    aggp, degp = _sc_aggregate(x, ii, with_deg=True)
    h = _tc_layer(aggp, degp, x, Wl1, bl1.reshape(1, D), Wr1, relu=True)
    (aggp2,) = _sc_aggregate(h, ii, with_deg=False)
    out = _tc_layer(aggp2, degp, h, Wl2, bl2.reshape(1, D), Wr2, relu=False)
    return out


# speedup vs baseline: 1.0569x; 1.0569x over previous
import dataclasses
import functools

import jax
import jax.numpy as jnp
from jax import lax
from jax.experimental import pallas as pl
from jax.experimental.pallas import tpu as pltpu
from jax.experimental.pallas import tpu_sc as plsc

N_NODES = 10000
N_EDGES = 320000
D = 128

NC = 2
NS = 16
NW = NC * NS
CHUNK = 128
GRP = 8
CPW = 80
EPW = CHUNK * CPW
E_PAD = EPW * NW
NBLK = CPW // GRP
N_PAD = 10112
RPW = N_PAD // NS
NZ = RPW // CHUNK
TAIL = RPW % CHUNK
DUMMY_ROW = N_NODES + 8

_MESH = plsc.VectorSubcoreMesh(core_axis_name="c", subcore_axis_name="s")


def _zero_stripe(zv, shared, base):
    for k in range(NZ):
        pltpu.sync_copy(zv, shared.at[pl.ds(base + k * CHUNK, CHUNK)])
    if TAIL:
        pltpu.sync_copy(zv.at[pl.ds(0, TAIL)],
                        shared.at[pl.ds(base + NZ * CHUNK, TAIL)])


def _sc_aggregate(x, ii):
    def body(x_hbm, ii_hbm, z128_hbm, agg_out,
             agg_sh, idx_v, r0, r1, sg0, sg1, ss0, ss1):
        c = lax.axis_index("c")
        s = lax.axis_index("s")
        w = c * NS + s
        base = s * RPW

        pltpu.sync_copy(z128_hbm, r0)
        _zero_stripe(r0, agg_sh, base)
        plsc.subcore_barrier()

        @pl.loop(0, NBLK)
        def _(t):
            pltpu.sync_copy(ii_hbm.at[w * NBLK + t], idx_v)
            for p0 in range(0, GRP, 2):
                p1 = p0 + 1
                g0 = pltpu.async_copy(x_hbm.at[idx_v.at[0, p0]], r0, sg0)
                g1 = pltpu.async_copy(x_hbm.at[idx_v.at[0, p1]], r1, sg1)
                g0.wait()
                s0 = pltpu.async_copy(r0, agg_sh.at[idx_v.at[1, p0]], ss0,
                                      add=True)
                g1.wait()
                s1 = pltpu.async_copy(r1, agg_sh.at[idx_v.at[1, p1]], ss1,
                                      add=True)
                s0.wait()
                s1.wait()

        plsc.subcore_barrier()
        pltpu.sync_copy(agg_sh.at[pl.ds(base, RPW)],
                        agg_out.at[pl.ds(c * N_PAD + base, RPW)])

    k = pl.kernel(
        body,
        out_type=jax.ShapeDtypeStruct((NC * N_PAD, D), jnp.float32),
        mesh=_MESH,
        scratch_types=[
            pltpu.VMEM_SHARED((N_PAD, D), jnp.float32),
            pltpu.VMEM((2, GRP, CHUNK), jnp.int32),
            pltpu.VMEM((CHUNK, D), jnp.float32),
            pltpu.VMEM((CHUNK, D), jnp.float32),
            pltpu.SemaphoreType.DMA,
            pltpu.SemaphoreType.DMA,
            pltpu.SemaphoreType.DMA,
            pltpu.SemaphoreType.DMA,
        ])
    return k(x, ii, jnp.zeros((CHUNK, D), jnp.float32))


def _sc_degree(di):
    def body(di_hbm, deg_out, hist, idx_d):
        c = lax.axis_index("c")
        s = lax.axis_index("s")
        w = c * NS + s

        zeros16 = jnp.zeros((16,), jnp.float32)
        ones16 = jnp.ones((16,), jnp.float32)

        @pl.loop(0, N_PAD // 16)
        def _(i):
            hist[pl.ds(i * 16, 16)] = zeros16

        @pl.loop(0, NBLK)
        def _(t):
            pltpu.sync_copy(di_hbm.at[w * NBLK + t], idx_d)
            for b in range(GRP):
                for j in range(CHUNK // 16):
                    idxv = idx_d[b, pl.ds(j * 16, 16)]
                    plsc.addupdate_scatter(hist, [idxv], ones16)

        pltpu.sync_copy(hist, deg_out.at[w])

    cp = pltpu.CompilerParams()
    if "needs_layout_passes" in pltpu.CompilerParams.__dataclass_fields__:
        cp = dataclasses.replace(cp, needs_layout_passes=False)
    k = pl.kernel(
        body,
        out_type=jax.ShapeDtypeStruct((NW, N_PAD), jnp.float32),
        mesh=_MESH,
        compiler_params=cp,
        scratch_types=[
            pltpu.VMEM((N_PAD,), jnp.float32),
            pltpu.VMEM((GRP, CHUNK), jnp.int32),
        ])
    return k(di)


def _tc_layer(aggp, degp, x, Wl, bl, Wr, relu):
    def body(aggp_ref, degp_ref, ones_ref, x_ref, wl_ref, bl_ref, wr_ref,
             o_ref):
        agg = aggp_ref[:N_NODES, :] + aggp_ref[N_PAD:N_PAD + N_NODES, :]
        deg = lax.dot_general(degp_ref[:, :N_NODES], ones_ref[...],
                              (((0,), (0,)), ((), ())),
                              preferred_element_type=jnp.float32)
        mean = agg / jnp.maximum(deg, 1.0)
        h = lax.dot_general(mean, wl_ref[...], (((1,), (1,)), ((), ())),
                            preferred_element_type=jnp.float32)
        h = h + bl_ref[...]
        h = h + lax.dot_general(x_ref[...], wr_ref[...],
                                (((1,), (1,)), ((), ())),
                                preferred_element_type=jnp.float32)
        o_ref[...] = jnp.maximum(h, 0.0) if relu else h

    return pl.pallas_call(
        body,
        out_shape=jax.ShapeDtypeStruct((N_NODES, D), jnp.float32),
    )(aggp, degp, jnp.ones((NW, 1), jnp.float32), x, Wl, bl, Wr)


@jax.jit
def kernel(x, edge_index, Wl1, bl1, Wr1, Wl2, bl2, Wr2):
    src = edge_index[0].astype(jnp.int32)
    dst = edge_index[1].astype(jnp.int32)
    pad = E_PAD - N_EDGES
    src = jnp.concatenate([src, jnp.zeros((pad,), jnp.int32)])
    dst = jnp.concatenate([dst, jnp.full((pad,), DUMMY_ROW, jnp.int32)])
    si = src.reshape(NW * NBLK, GRP, CHUNK)
    di = dst.reshape(NW * NBLK, GRP, CHUNK)
    ii = jnp.stack([si, di], axis=1)

    degp = _sc_degree(di)
    aggp = _sc_aggregate(x, ii)
    h = _tc_layer(aggp, degp, x, Wl1, bl1.reshape(1, D), Wr1, relu=True)
    aggp2 = _sc_aggregate(h, ii)
    out = _tc_layer(aggp2, degp, h, Wl2, bl2.reshape(1, D), Wr2, relu=False)
    return out
